# baseline (speedup 1.0000x reference)
import functools

import jax
import jax.numpy as jnp
from jax import lax
from jax.experimental import pallas as pl
from jax.experimental.pallas import tpu as pltpu
from jax.experimental.pallas import tpu_sc as plsc

VOCAB = 100000
EMB = 128
HID = 1024
TAGS = 256
B = 4096
L = 200

NC = 2
NS = 16
NW = NC * NS
RPW = B // NW
TPW = RPW * L
NVEC = EMB // 16
PK = EMB // 2
NBUF = 4
S0, S1 = 128, 72
INV_L = 1.0 / L


def _pool_body(x_hbm, emb_hbm, out_hbm, idx_all, rows_v, out_v, sems):
    wid = lax.axis_index("s") * NC + lax.axis_index("c")

    pltpu.sync_copy(x_hbm.at[pl.ds(wid * TPW, TPW)], idx_all)

    def fire(row, buf):
        o = pl.multiple_of(row * L, 8)
        pltpu.async_copy(emb_hbm.at[idx_all.at[pl.ds(o, S0)]],
                         rows_v.at[buf, pl.ds(0, S0)], sems.at[buf])
        pltpu.async_copy(emb_hbm.at[idx_all.at[pl.ds(o + S0, S1)]],
                         rows_v.at[buf, pl.ds(S0, S1)], sems.at[buf])

    def drain(buf):
        pltpu.make_async_copy(emb_hbm.at[idx_all.at[pl.ds(0, S0)]],
                              rows_v.at[buf, pl.ds(0, S0)], sems.at[buf]).wait()
        pltpu.make_async_copy(emb_hbm.at[idx_all.at[pl.ds(0, S1)]],
                              rows_v.at[buf, pl.ds(S0, S1)], sems.at[buf]).wait()

    for r in range(NBUF - 1):
        fire(r, r)

    def outer(i):
        for b in range(NBUF):
            row = i + b
            fire(lax.min(row + NBUF - 1, RPW - 1), (b + NBUF - 1) % NBUF)
            drain(b)

            def red(j, accs):
                accs = list(accs)
                for k in range(PK // 16):
                    w = rows_v[b, j, pl.ds(16 * k, 16)]
                    lo = lax.bitcast_convert_type(
                        lax.shift_left(w, 16), jnp.float32)
                    hi = lax.bitcast_convert_type(
                        lax.bitwise_and(w, jnp.int32(-65536)), jnp.float32)
                    accs[k] = accs[k] + lo
                    accs[k + 4] = accs[k + 4] + hi
                return tuple(accs)

            accs = lax.fori_loop(
                0, L, red,
                tuple(jnp.zeros((16,), jnp.float32) for _ in range(NVEC)))
            for k in range(NVEC):
                out_v[row, pl.ds(16 * k, 16)] = accs[k] * INV_L

    pl.loop(0, RPW, step=NBUF)(outer)
    for b in range(NBUF - 1):
        drain(b)
    pltpu.sync_copy(out_v, out_hbm.at[pl.ds(wid * RPW, RPW)])


@functools.partial(jax.jit, static_argnames=())
def _pool(x_flat, packed):
    mesh = plsc.VectorSubcoreMesh(core_axis_name="c", subcore_axis_name="s")
    return pl.kernel(
        _pool_body,
        out_type=jax.ShapeDtypeStruct((B, EMB), jnp.float32),
        mesh=mesh,
        compiler_params=pltpu.CompilerParams(use_tc_tiling_on_sc=False),
        scratch_types=[
            pltpu.VMEM((TPW,), jnp.int32),
            pltpu.VMEM((NBUF, L, PK), jnp.int32),
            pltpu.VMEM((RPW, EMB), jnp.float32),
            pltpu.SemaphoreType.DMA((NBUF,)),
        ],
    )(x_flat, packed)


def _mlp_block(p_ref, w1_ref, b1_ref, w2_ref, b2_ref, o_ref):
    h = jnp.dot(p_ref[...], w1_ref[...], preferred_element_type=jnp.float32)
    h = jnp.maximum(h + b1_ref[...], 0.0)
    o_ref[...] = jnp.dot(h, w2_ref[...],
                         preferred_element_type=jnp.float32) + b2_ref[...]


def _mlp(pooled, W1, b1, W2, b2):
    BM = 512
    return pl.pallas_call(
        _mlp_block,
        grid=(B // BM,),
        in_specs=[
            pl.BlockSpec((BM, EMB), lambda i: (i, 0)),
            pl.BlockSpec((EMB, HID), lambda i: (0, 0)),
            pl.BlockSpec((1, HID), lambda i: (0, 0)),
            pl.BlockSpec((HID, TAGS), lambda i: (0, 0)),
            pl.BlockSpec((1, TAGS), lambda i: (0, 0)),
        ],
        out_specs=pl.BlockSpec((BM, TAGS), lambda i: (i, 0)),
        out_shape=jax.ShapeDtypeStruct((B, TAGS), jnp.float32),
    )(pooled, W1, b1.reshape(1, HID), W2, b2.reshape(1, TAGS))


def kernel(x, emb, W1, b1, W2, b2):
    x_flat = x.astype(jnp.int32).reshape(B * L)
    u = lax.bitcast_convert_type(emb, jnp.uint32)
    r = (u + jnp.uint32(0x7FFF) + ((u >> 16) & jnp.uint32(1))) >> 16
    packed = lax.bitcast_convert_type(r[:, :PK] | (r[:, PK:] << 16), jnp.int32)
    pooled = _pool(x_flat, packed)
    return _mlp(pooled, W1, b1, W2, b2)


# speedup vs baseline: 1.1056x; 1.1056x over previous
import functools

import jax
import jax.numpy as jnp
from jax import lax
from jax.experimental import pallas as pl
from jax.experimental.pallas import tpu as pltpu
from jax.experimental.pallas import tpu_sc as plsc

VOCAB = 100000
EMB = 128
HID = 1024
TAGS = 256
B = 4096
L = 200

NC = 2
NS = 16
NW = NC * NS
RPW = B // NW
TPW = RPW * L
NVEC = EMB // 16
PK = EMB // 2
NBUF = 4
S0, S1 = 128, 72
INV_L = 1.0 / L

VPW = VOCAB // NW
CH = 125
NCH = VPW // CH


def _pack_body(emb_hbm, out_hbm, in_v, out_v, isem, osem):
    wid = lax.axis_index("s") * NC + lax.axis_index("c")
    base = wid * VPW

    def fire_in(c, s):
        pltpu.async_copy(emb_hbm.at[pl.ds((base + c * CH) * EMB, CH * EMB)],
                         in_v.at[s], isem.at[s])

    fire_in(0, 0)
    for c in range(NCH):
        s = c % 2
        if c + 1 < NCH:
            fire_in(c + 1, 1 - s)
        pltpu.make_async_copy(emb_hbm.at[pl.ds(0, CH * EMB)],
                              in_v.at[s], isem.at[s]).wait()
        if c >= 2:
            pltpu.make_async_copy(out_v.at[s],
                                  out_hbm.at[pl.ds(0, CH)], osem.at[s]).wait()

        def body(r, _):
            o = pl.multiple_of(r * EMB, 8)
            for k in range(PK // 16):
                u0 = lax.bitcast_convert_type(
                    in_v[s, pl.ds(o + 16 * k, 16)], jnp.uint32)
                u1 = lax.bitcast_convert_type(
                    in_v[s, pl.ds(o + PK + 16 * k, 16)], jnp.uint32)
                r0 = (u0 + jnp.uint32(0x7FFF)
                      + ((u0 >> 16) & jnp.uint32(1))) >> 16
                r1 = (u1 + jnp.uint32(0x7FFF)
                      + ((u1 >> 16) & jnp.uint32(1))) >> 16
                out_v[s, r, pl.ds(16 * k, 16)] = lax.bitcast_convert_type(
                    r0 | (r1 << 16), jnp.int32)
            return 0

        lax.fori_loop(0, CH, body, 0)
        pltpu.async_copy(out_v.at[s],
                         out_hbm.at[pl.ds(base + c * CH, CH)], osem.at[s])
    for s in ((NCH - 2) % 2, (NCH - 1) % 2):
        pltpu.make_async_copy(out_v.at[s],
                              out_hbm.at[pl.ds(0, CH)], osem.at[s]).wait()


def _pool_body(x_hbm, emb_hbm, out_hbm, idx_all, rows_v, out_v, sems):
    wid = lax.axis_index("s") * NC + lax.axis_index("c")

    pltpu.sync_copy(x_hbm.at[pl.ds(wid * TPW, TPW)], idx_all)

    def fire(row, buf):
        o = pl.multiple_of(row * L, 8)
        pltpu.async_copy(emb_hbm.at[idx_all.at[pl.ds(o, S0)]],
                         rows_v.at[buf, pl.ds(0, S0)], sems.at[buf])
        pltpu.async_copy(emb_hbm.at[idx_all.at[pl.ds(o + S0, S1)]],
                         rows_v.at[buf, pl.ds(S0, S1)], sems.at[buf])

    def drain(buf):
        pltpu.make_async_copy(emb_hbm.at[idx_all.at[pl.ds(0, S0)]],
                              rows_v.at[buf, pl.ds(0, S0)], sems.at[buf]).wait()
        pltpu.make_async_copy(emb_hbm.at[idx_all.at[pl.ds(0, S1)]],
                              rows_v.at[buf, pl.ds(S0, S1)], sems.at[buf]).wait()

    for r in range(NBUF - 1):
        fire(r, r)

    def outer(i):
        for b in range(NBUF):
            row = i + b
            fire(lax.min(row + NBUF - 1, RPW - 1), (b + NBUF - 1) % NBUF)
            drain(b)

            def red(j, accs):
                accs = list(accs)
                for k in range(PK // 16):
                    w = rows_v[b, j, pl.ds(16 * k, 16)]
                    lo = lax.bitcast_convert_type(
                        lax.shift_left(w, 16), jnp.float32)
                    hi = lax.bitcast_convert_type(
                        lax.bitwise_and(w, jnp.int32(-65536)), jnp.float32)
                    accs[k] = accs[k] + lo
                    accs[k + 4] = accs[k + 4] + hi
                return tuple(accs)

            accs = lax.fori_loop(
                0, L, red,
                tuple(jnp.zeros((16,), jnp.float32) for _ in range(NVEC)))
            for k in range(NVEC):
                out_v[row, pl.ds(16 * k, 16)] = accs[k] * INV_L

    pl.loop(0, RPW, step=NBUF)(outer)
    for b in range(NBUF - 1):
        drain(b)
    pltpu.sync_copy(out_v, out_hbm.at[pl.ds(wid * RPW, RPW)])


@functools.partial(jax.jit, static_argnames=())
def _pool(x_flat, emb_flat):
    mesh = plsc.VectorSubcoreMesh(core_axis_name="c", subcore_axis_name="s")
    packed = pl.kernel(
        _pack_body,
        out_type=jax.ShapeDtypeStruct((VOCAB, PK), jnp.int32),
        mesh=mesh,
        compiler_params=pltpu.CompilerParams(use_tc_tiling_on_sc=False),
        scratch_types=[
            pltpu.VMEM((2, CH * EMB), jnp.float32),
            pltpu.VMEM((2, CH, PK), jnp.int32),
            pltpu.SemaphoreType.DMA((2,)),
            pltpu.SemaphoreType.DMA((2,)),
        ],
    )(emb_flat)
    return pl.kernel(
        _pool_body,
        out_type=jax.ShapeDtypeStruct((B, EMB), jnp.float32),
        mesh=mesh,
        compiler_params=pltpu.CompilerParams(use_tc_tiling_on_sc=False),
        scratch_types=[
            pltpu.VMEM((TPW,), jnp.int32),
            pltpu.VMEM((NBUF, L, PK), jnp.int32),
            pltpu.VMEM((RPW, EMB), jnp.float32),
            pltpu.SemaphoreType.DMA((NBUF,)),
        ],
    )(x_flat, packed)


def _mlp_block(p_ref, w1_ref, b1_ref, w2_ref, b2_ref, o_ref):
    h = jnp.dot(p_ref[...], w1_ref[...], preferred_element_type=jnp.float32)
    h = jnp.maximum(h + b1_ref[...], 0.0)
    o_ref[...] = jnp.dot(h, w2_ref[...],
                         preferred_element_type=jnp.float32) + b2_ref[...]


def _mlp(pooled, W1, b1, W2, b2):
    BM = 512
    return pl.pallas_call(
        _mlp_block,
        grid=(B // BM,),
        in_specs=[
            pl.BlockSpec((BM, EMB), lambda i: (i, 0)),
            pl.BlockSpec((EMB, HID), lambda i: (0, 0)),
            pl.BlockSpec((1, HID), lambda i: (0, 0)),
            pl.BlockSpec((HID, TAGS), lambda i: (0, 0)),
            pl.BlockSpec((1, TAGS), lambda i: (0, 0)),
        ],
        out_specs=pl.BlockSpec((BM, TAGS), lambda i: (i, 0)),
        out_shape=jax.ShapeDtypeStruct((B, TAGS), jnp.float32),
    )(pooled, W1, b1.reshape(1, HID), W2, b2.reshape(1, TAGS))


def kernel(x, emb, W1, b1, W2, b2):
    x_flat = x.astype(jnp.int32).reshape(B * L)
    emb_flat = emb.reshape(VOCAB * EMB)
    pooled = _pool(x_flat, emb_flat)
    return _mlp(pooled, W1, b1, W2, b2)


# speedup vs baseline: 1.3546x; 1.2252x over previous
import functools

import jax
import jax.numpy as jnp
from jax import lax
from jax.experimental import pallas as pl
from jax.experimental.pallas import tpu as pltpu
from jax.experimental.pallas import tpu_sc as plsc

VOCAB = 100000
EMB = 128
HID = 1024
TAGS = 256
B = 4096
L = 200

NC = 2
NS = 16
NW = NC * NS
RPW = B // NW
TPW = RPW * L
NVEC = EMB // 16
PK = EMB // 2
NBUF = 4
S0, S1 = 128, 72
INV_L = 1.0 / L


def _pool_body(x_hbm, emb_hbm, out_hbm, idx_all, rows_v, out_v, sems, osems):
    wid = lax.axis_index("s") * NC + lax.axis_index("c")

    pltpu.sync_copy(x_hbm.at[pl.ds(wid * TPW, TPW)], idx_all)

    def fire(row, buf):
        o = pl.multiple_of(row * L, 8)
        pltpu.async_copy(emb_hbm.at[idx_all.at[pl.ds(o, S0)]],
                         rows_v.at[buf, pl.ds(0, S0)], sems.at[buf])
        pltpu.async_copy(emb_hbm.at[idx_all.at[pl.ds(o + S0, S1)]],
                         rows_v.at[buf, pl.ds(S0, S1)], sems.at[buf])

    def drain(buf):
        pltpu.make_async_copy(emb_hbm.at[idx_all.at[pl.ds(0, S0)]],
                              rows_v.at[buf, pl.ds(0, S0)], sems.at[buf]).wait()
        pltpu.make_async_copy(emb_hbm.at[idx_all.at[pl.ds(0, S1)]],
                              rows_v.at[buf, pl.ds(S0, S1)], sems.at[buf]).wait()

    base_row = wid * RPW
    for r in range(NBUF - 1):
        fire(r, r)
    for s in range(NBUF):
        pltpu.async_copy(out_hbm.at[pl.ds(base_row, 1)], out_v.at[s],
                         osems.at[s])

    def outer(i):
        for b in range(NBUF):
            row = i + b
            fire(lax.min(row + NBUF - 1, RPW - 1), (b + NBUF - 1) % NBUF)
            drain(b)

            def red(j, accs):
                return tuple(a + rows_v[b, j, pl.ds(16 * k, 16)]
                             for k, a in enumerate(accs))

            accs = lax.fori_loop(
                0, L, red,
                tuple(jnp.zeros((16,), jnp.float32) for _ in range(NVEC)))
            pltpu.make_async_copy(out_v.at[b], out_hbm.at[pl.ds(base_row, 1)],
                                  osems.at[b]).wait()
            for k in range(NVEC):
                out_v[b, 0, pl.ds(16 * k, 16)] = accs[k] * INV_L
            pltpu.async_copy(out_v.at[b], out_hbm.at[pl.ds(base_row + row, 1)],
                             osems.at[b])

    pl.loop(0, RPW, step=NBUF)(outer)
    for b in range(NBUF - 1):
        drain(b)
    for s in range(NBUF):
        pltpu.make_async_copy(out_v.at[s], out_hbm.at[pl.ds(base_row, 1)],
                              osems.at[s]).wait()


@functools.partial(jax.jit, static_argnames=())
def _pool(x_flat, packed):
    mesh = plsc.VectorSubcoreMesh(core_axis_name="c", subcore_axis_name="s")
    return pl.kernel(
        _pool_body,
        out_type=jax.ShapeDtypeStruct((B, EMB), jnp.float32),
        mesh=mesh,
        scratch_types=[
            pltpu.VMEM((TPW,), jnp.int32),
            pltpu.VMEM((NBUF, L, EMB), jnp.float32),
            pltpu.VMEM((NBUF, 1, EMB), jnp.float32),
            pltpu.SemaphoreType.DMA((NBUF,)),
            pltpu.SemaphoreType.DMA((NBUF,)),
        ],
    )(x_flat, packed)


def _mlp_block(p_ref, w1_ref, b1_ref, w2_ref, b2_ref, o_ref):
    h = jnp.dot(p_ref[...], w1_ref[...], preferred_element_type=jnp.float32)
    h = jnp.maximum(h + b1_ref[...], 0.0)
    o_ref[...] = jnp.dot(h, w2_ref[...],
                         preferred_element_type=jnp.float32) + b2_ref[...]


def _mlp(pooled, W1, b1, W2, b2):
    BM = 512
    return pl.pallas_call(
        _mlp_block,
        grid=(B // BM,),
        in_specs=[
            pl.BlockSpec((BM, EMB), lambda i: (i, 0)),
            pl.BlockSpec((EMB, HID), lambda i: (0, 0)),
            pl.BlockSpec((1, HID), lambda i: (0, 0)),
            pl.BlockSpec((HID, TAGS), lambda i: (0, 0)),
            pl.BlockSpec((1, TAGS), lambda i: (0, 0)),
        ],
        out_specs=pl.BlockSpec((BM, TAGS), lambda i: (i, 0)),
        out_shape=jax.ShapeDtypeStruct((B, TAGS), jnp.float32),
    )(pooled, W1, b1.reshape(1, HID), W2, b2.reshape(1, TAGS))


def kernel(x, emb, W1, b1, W2, b2):
    x_flat = x.astype(jnp.int32).reshape(B * L)
    pooled = _pool(x_flat, emb)
    return _mlp(pooled, W1, b1, W2, b2)


# speedup vs baseline: 1.4165x; 1.0457x over previous
import functools

import jax
import jax.numpy as jnp
from jax import lax
from jax.experimental import pallas as pl
from jax.experimental.pallas import tpu as pltpu
from jax.experimental.pallas import tpu_sc as plsc

VOCAB = 100000
EMB = 128
HID = 1024
TAGS = 256
B = 4096
L = 200

NC = 2
NS = 16
NW = NC * NS
RPW = B // NW
TPW = RPW * L
NVEC = EMB // 16
PK = EMB // 2
NBUF = 3
S0, S1 = 128, 72
INV_L = 1.0 / L


def _pool_body(x_hbm, emb_hbm, out_hbm, idx_all, rows_v, out_v, sems):
    wid = lax.axis_index("s") * NC + lax.axis_index("c")

    pltpu.sync_copy(x_hbm.at[pl.ds(wid * RPW, RPW)], idx_all)

    def fire(row, buf):
        pltpu.async_copy(emb_hbm.at[idx_all.at[row, pl.ds(0, S0)]],
                         rows_v.at[buf, pl.ds(0, S0)], sems.at[buf])
        pltpu.async_copy(emb_hbm.at[idx_all.at[row, pl.ds(S0, S1)]],
                         rows_v.at[buf, pl.ds(S0, S1)], sems.at[buf])

    def drain(buf):
        pltpu.make_async_copy(emb_hbm.at[idx_all.at[0, pl.ds(0, S0)]],
                              rows_v.at[buf, pl.ds(0, S0)], sems.at[buf]).wait()
        pltpu.make_async_copy(emb_hbm.at[idx_all.at[0, pl.ds(S0, S1)]],
                              rows_v.at[buf, pl.ds(S0, S1)], sems.at[buf]).wait()

    for r in range(NBUF - 1):
        fire(r, r)

    def outer(i):
        for b in range(NBUF):
            row = i + b
            fire(lax.min(row + NBUF - 1, RPW - 1), (b + NBUF - 1) % NBUF)
            drain(b)

            def red(j, accs):
                return tuple(a + rows_v[b, j, pl.ds(16 * k, 16)]
                             for k, a in enumerate(accs))

            accs = lax.fori_loop(
                0, L, red,
                tuple(jnp.zeros((16,), jnp.float32) for _ in range(NVEC)))
            for k in range(NVEC):
                out_v[row, pl.ds(16 * k, 16)] = accs[k] * INV_L

    pl.loop(0, RPW, step=NBUF)(outer)
    for b in range(NBUF - 1):
        drain(b)
    pltpu.sync_copy(out_v, out_hbm.at[pl.ds(wid * RPW, RPW)])


@functools.partial(jax.jit, static_argnames=())
def _pool(x_flat, packed):
    mesh = plsc.VectorSubcoreMesh(core_axis_name="c", subcore_axis_name="s")
    return pl.kernel(
        _pool_body,
        out_type=jax.ShapeDtypeStruct((B, EMB), jnp.float32),
        mesh=mesh,
        scratch_types=[
            pltpu.VMEM((RPW, L), jnp.int32),
            pltpu.VMEM((NBUF, L, EMB), jnp.float32),
            pltpu.VMEM((RPW, EMB), jnp.float32),
            pltpu.SemaphoreType.DMA((NBUF,)),
        ],
    )(x_flat, packed)


def _mlp_block(p_ref, w1_ref, b1_ref, w2_ref, b2_ref, o_ref):
    h = jnp.dot(p_ref[...], w1_ref[...], preferred_element_type=jnp.float32)
    h = jnp.maximum(h + b1_ref[...], 0.0)
    o_ref[...] = jnp.dot(h, w2_ref[...],
                         preferred_element_type=jnp.float32) + b2_ref[...]


def _mlp(pooled, W1, b1, W2, b2):
    BM = 512
    return pl.pallas_call(
        _mlp_block,
        grid=(B // BM,),
        in_specs=[
            pl.BlockSpec((BM, EMB), lambda i: (i, 0)),
            pl.BlockSpec((EMB, HID), lambda i: (0, 0)),
            pl.BlockSpec((1, HID), lambda i: (0, 0)),
            pl.BlockSpec((HID, TAGS), lambda i: (0, 0)),
            pl.BlockSpec((1, TAGS), lambda i: (0, 0)),
        ],
        out_specs=pl.BlockSpec((BM, TAGS), lambda i: (i, 0)),
        out_shape=jax.ShapeDtypeStruct((B, TAGS), jnp.float32),
    )(pooled, W1, b1.reshape(1, HID), W2, b2.reshape(1, TAGS))


def kernel(x, emb, W1, b1, W2, b2):
    pooled = _pool(x.astype(jnp.int32), emb)
    return _mlp(pooled, W1, b1, W2, b2)
